# Initial kernel scaffold; baseline (speedup 1.0000x reference)
import jax
import jax.numpy as jnp
from jax.experimental import pallas as pl


def kernel(mem_weak, mem_strong, mem_label, mem_partial, mem_task, mem_index, sample_weak, sample_strong, label, partial, task, index, rand_idx):
    raise NotImplementedError("write your pallas kernel here")


# speedup vs baseline: 1.0570x; 1.0570x over previous
import jax
import jax.numpy as jnp
from jax import lax
from jax.experimental import pallas as pl
from jax.experimental.pallas import tpu as pltpu
from jax.experimental.pallas import tpu_sc as plsc

MEM = 10000
B = 2048
NCL = 100
D = 3 * 32 * 32
PK = 128
NC = 2
NS = 16
NT = NC * NS
GR = 16
NGROUPS = MEM // GR
CHUNK = 16
LMAX = B + NT * CHUNK


def _extract(vmem64, j):
    return vmem64[pl.ds(j, 1)][0]


def _body(mw, ms, mp, sw, ss, bp_in, li_l, lm_l, meta,
          ow, os_, op_,
          bw, bs, bp, idxb, idxm, vmeta, rsem, wsem):
    c = lax.axis_index("c")
    s = lax.axis_index("s")
    w = s * NC + c

    ng = (NGROUPS - w + NT - 1) // NT

    def p1(i, carry):
        r0 = (i * NT + w) * GR
        rds = (
            pltpu.async_copy(mw.at[pl.ds(r0, GR)], bw, rsem),
            pltpu.async_copy(ms.at[pl.ds(r0, GR)], bs, rsem),
            pltpu.async_copy(mp.at[pl.ds(r0, GR)], bp, rsem),
        )
        for cp in rds:
            cp.wait()
        wrs = (
            pltpu.async_copy(bw, ow.at[pl.ds(r0, GR)], wsem),
            pltpu.async_copy(bs, os_.at[pl.ds(r0, GR)], wsem),
            pltpu.async_copy(bp, op_.at[pl.ds(r0, GR)], wsem),
        )
        for cp in wrs:
            cp.wait()
        return carry

    lax.fori_loop(0, ng, p1, 0)

    pltpu.sync_copy(meta, vmeta)
    start_e = _extract(vmeta, w)
    nch = _extract(vmeta, NT + w)

    def p2(i, carry):
        e0 = pl.multiple_of(start_e + i * CHUNK, CHUNK)
        pltpu.sync_copy(li_l.at[pl.ds(e0, CHUNK)], idxb)
        pltpu.sync_copy(lm_l.at[pl.ds(e0, CHUNK)], idxm)
        gts = (
            pltpu.async_copy(sw.at[idxb], bw, rsem),
            pltpu.async_copy(ss.at[idxb], bs, rsem),
            pltpu.async_copy(bp_in.at[idxb], bp, rsem),
        )
        for cp in gts:
            cp.wait()
        sts = (
            pltpu.async_copy(bw, ow.at[idxm], wsem),
            pltpu.async_copy(bs, os_.at[idxm], wsem),
            pltpu.async_copy(bp, op_.at[idxm], wsem),
        )
        for cp in sts:
            cp.wait()
        return carry

    lax.fori_loop(0, nch, p2, 0)


def kernel(mem_weak, mem_strong, mem_label, mem_partial, mem_task, mem_index,
           sample_weak, sample_strong, label, partial, task, index, rand_idx):
    i32 = jnp.int32
    f32 = jnp.float32
    mw2 = mem_weak.reshape(MEM, D)
    ms2 = mem_strong.reshape(MEM, D)
    sw2 = sample_weak.reshape(B, D)
    ss2 = sample_strong.reshape(B, D)

    mp_pad = jnp.concatenate(
        [mem_partial, mem_label[:, None], mem_task[:, None],
         mem_index[:, None], jnp.zeros((MEM, PK - NCL - 3), i32)], axis=1)
    taskcol = jnp.full((B,), task, i32)
    bp_pad = jnp.concatenate(
        [partial, label[:, None], taskcol[:, None],
         index[:, None], jnp.zeros((B, PK - NCL - 3), i32)], axis=1)

    ii = jnp.arange(B, dtype=i32)
    win = jnp.full((MEM,), -1, i32).at[rand_idx].max(ii, mode="drop")
    safe = jnp.where(rand_idx < MEM, rand_idx, 0)
    is_win = (rand_idx < MEM) & (win[safe] == ii)
    owner = jnp.where(is_win, (rand_idx // GR) % NT, NT).astype(i32)
    ordr = jnp.argsort(owner, stable=True)
    si = ii[ordr]
    sm = jnp.where(is_win, rand_idx, 0).astype(i32)[ordr]
    cnt = jnp.bincount(owner, length=NT + 1)[:NT].astype(i32)
    cntp = ((cnt + CHUNK - 1) // CHUNK) * CHUNK
    zero1 = jnp.zeros((1,), i32)
    start = jnp.concatenate([zero1, jnp.cumsum(cntp)[:-1].astype(i32)])
    rawstart = jnp.concatenate([zero1, jnp.cumsum(cnt)[:-1].astype(i32)])
    pos = jnp.arange(LMAX, dtype=i32)
    bkt = jnp.searchsorted(start, pos, side="right").astype(i32) - 1
    off = pos - start[bkt]
    src = rawstart[bkt] + jnp.minimum(off, jnp.maximum(cnt[bkt] - 1, 0))
    src = jnp.clip(src, 0, B - 1)
    li_l = si[src]
    lm_l = sm[src]
    meta = jnp.concatenate([start, cntp // CHUNK]).astype(i32)

    ow, os_, op_ = pl.kernel(
        _body,
        out_type=[
            jax.ShapeDtypeStruct((MEM, D), f32),
            jax.ShapeDtypeStruct((MEM, D), f32),
            jax.ShapeDtypeStruct((MEM, PK), i32),
        ],
        mesh=plsc.VectorSubcoreMesh(core_axis_name="c", subcore_axis_name="s"),
        scratch_types=[
            pltpu.VMEM((GR, D), f32),
            pltpu.VMEM((GR, D), f32),
            pltpu.VMEM((GR, PK), i32),
            pltpu.VMEM((CHUNK,), i32),
            pltpu.VMEM((CHUNK,), i32),
            pltpu.VMEM((64,), i32),
            pltpu.SemaphoreType.DMA,
            pltpu.SemaphoreType.DMA,
        ],
    )(mw2, ms2, mp_pad, sw2, ss2, bp_pad, li_l, lm_l, meta)

    return (ow.reshape(mem_weak.shape), os_.reshape(mem_strong.shape),
            op_[:, NCL], op_[:, :NCL], op_[:, NCL + 1], op_[:, NCL + 2])
